# baseline (speedup 1.0000x reference)
import functools

import jax
import jax.numpy as jnp
from jax import lax
from jax.experimental import pallas as pl
from jax.experimental.pallas import tpu as pltpu
from jax.experimental.pallas import tpu_sc as plsc

_NUM_REQ = 512
_LITS = 4
_NC = 2
_NS = 16
_NW = _NC * _NS
_LANES = 16
_CHUNK = 8
_NGROUPS = _NUM_REQ // _LANES

_ONE_ONE = 0x3F803F80
_NEG_NEG = 0xBF80BF80 - 0x100000000


def _body(pack_h, preq_h, pvar_h, mreq_h, mvar_h, out_h,
          preq_v, pvar_v, mreq_v, mvar_v,
          var_v, scale_v, off_v, rows_a, rows_b, sem_a, sem_b, acc_v,
          prows_per_w, num_vars, n_plus, n_minus):
    c = lax.axis_index("c")
    s = lax.axis_index("s")
    wid = s * _NC + c
    base = wid * prows_per_w
    nchunks = prows_per_w // _CHUNK

    if n_plus:
        pltpu.sync_copy(preq_h, preq_v)
        pltpu.sync_copy(pvar_h, pvar_v)
    if n_minus:
        pltpu.sync_copy(mreq_h, mreq_v)
        pltpu.sync_copy(mvar_h, mvar_v)

    iota = lax.iota(jnp.int32, _LANES)
    zero_i = jnp.zeros((_LANES,), jnp.int32)

    def prep(req_v, varr_v, n, is_minus):
        ngrp = (n + _LANES - 1) // _LANES
        nm1 = jnp.full((_LANES,), n - 1, jnp.int32)
        nsplat = jnp.full((_LANES,), n, jnp.int32)
        sc_c = jnp.full((_LANES,), _NEG_NEG if is_minus else _ONE_ONE,
                        jnp.int32)
        of_c = jnp.full((_LANES,), _ONE_ONE if is_minus else 0, jnp.int32)

        def gbody(g, carry):
            pos = iota + g * _LANES
            cur_i = jnp.minimum(pos, nm1)
            req = plsc.load_gather(req_v, [cur_i])
            var = plsc.load_gather(varr_v, [cur_i])
            k = zero_i
            for t in (1, 2, 3):
                if is_minus:
                    q = pos + t
                    nb_i = jnp.minimum(q, nm1)
                    valid = q < nsplat
                else:
                    q = pos - t
                    nb_i = jnp.maximum(q, zero_i)
                    valid = q >= zero_i
                nb = plsc.load_gather(req_v, [nb_i])
                k = k + jnp.where(valid & (nb == req), 1, 0)
            if is_minus:
                k = 3 - k
            slot = k * _NUM_REQ + req
            mask = pos < nsplat
            plsc.store_scatter(var_v, [slot], var, mask=mask)
            plsc.store_scatter(scale_v, [slot], sc_c, mask=mask)
            plsc.store_scatter(off_v, [slot], of_c, mask=mask)
            return carry

        lax.fori_loop(0, ngrp, gbody, 0)

    if n_plus:
        prep(preq_v, pvar_v, n_plus, False)
    if n_minus:
        prep(mreq_v, mvar_v, n_minus, True)

    def make_g_body(rows_v):
        def g_body(g, accs):
            o = g * _LANES
            idx = [var_v[pl.ds(k * _NUM_REQ + o, _LANES)]
                   for k in range(_LITS)]
            sc = [plsc.bitcast(scale_v[pl.ds(k * _NUM_REQ + o, _LANES)],
                               jnp.bfloat16) for k in range(_LITS)]
            of = [plsc.bitcast(off_v[pl.ds(k * _NUM_REQ + o, _LANES)],
                               jnp.bfloat16) for k in range(_LITS)]
            accs = list(accs)
            for r in range(_CHUNK):
                rsplat = jnp.full((_LANES,), r, jnp.int32)
                vals = [of[k] + sc[k] * plsc.bitcast(
                            plsc.load_gather(rows_v, [rsplat, idx[k]]),
                            jnp.bfloat16)
                        for k in range(_LITS)]
                m = jnp.maximum(jnp.maximum(vals[0], vals[1]),
                                jnp.maximum(vals[2], vals[3]))
                a, b = plsc.unpack(m, format=plsc.PackFormat.INTERLEAVED,
                                   preferred_element_type=jnp.float32)
                accs[(2 * r) % 4] = accs[(2 * r) % 4] + a
                accs[(2 * r + 1) % 4] = accs[(2 * r + 1) % 4] + b
            return tuple(accs)
        return g_body

    def start(ci, dst, sem):
        pltpu.async_copy(pack_h.at[pl.ds(base + ci * _CHUNK, _CHUNK)],
                         dst, sem)

    def wait(dst, sem):
        pltpu.make_async_copy(pack_h.at[pl.ds(0, _CHUNK)], dst, sem).wait()

    last = nchunks - 1
    start(0, rows_a, sem_a)

    def pair_body(p, accs):
        ci1 = 2 * p + 1
        ci2 = jnp.minimum(2 * p + 2, last)
        start(ci1, rows_b, sem_b)
        wait(rows_a, sem_a)
        accs = lax.fori_loop(0, _NGROUPS, make_g_body(rows_a), accs)
        start(ci2, rows_a, sem_a)
        wait(rows_b, sem_b)
        accs = lax.fori_loop(0, _NGROUPS, make_g_body(rows_b), accs)
        return accs

    z = jnp.zeros((_LANES,), jnp.float32)
    accs = lax.fori_loop(0, nchunks // 2, pair_body, (z, z, z, z))
    wait(rows_a, sem_a)
    acc_v[...] = accs[0] + accs[1] + accs[2] + accs[3]
    pltpu.sync_copy(acc_v, out_h.at[wid])


def kernel(preds, plus_req, plus_var, minus_req, minus_var):
    batch, num_vars = preds.shape
    n_plus = plus_req.shape[0]
    n_minus = minus_req.shape[0]
    tab = _LITS * _NUM_REQ

    u = jax.lax.bitcast_convert_type(preds, jnp.uint32)
    a, b = u[0::2], u[1::2]

    def _rne_hi(x):
        return (x + jnp.uint32(0x7FFF) + ((x >> 16) & jnp.uint32(1))) >> 16

    packed = jax.lax.bitcast_convert_type(
        (_rne_hi(b) << 16) | _rne_hi(a), jnp.int32)

    prows_per_w = (batch // 2) // _NW
    mesh = plsc.VectorSubcoreMesh(core_axis_name="c", subcore_axis_name="s")
    sc_call = functools.partial(
        pl.kernel,
        out_type=jax.ShapeDtypeStruct((_NW, _LANES), jnp.float32),
        mesh=mesh,
        compiler_params=pltpu.CompilerParams(needs_layout_passes=False),
        scratch_types=[
            pltpu.VMEM((max(n_plus, 1),), jnp.int32),
            pltpu.VMEM((max(n_plus, 1),), jnp.int32),
            pltpu.VMEM((max(n_minus, 1),), jnp.int32),
            pltpu.VMEM((max(n_minus, 1),), jnp.int32),
            pltpu.VMEM((tab,), jnp.int32),
            pltpu.VMEM((tab,), jnp.int32),
            pltpu.VMEM((tab,), jnp.int32),
            pltpu.VMEM((_CHUNK, num_vars), jnp.int32),
            pltpu.VMEM((_CHUNK, num_vars), jnp.int32),
            pltpu.SemaphoreType.DMA,
            pltpu.SemaphoreType.DMA,
            pltpu.VMEM((_LANES,), jnp.float32),
        ],
    )(functools.partial(_body, prows_per_w=prows_per_w, num_vars=num_vars,
                        n_plus=n_plus, n_minus=n_minus))

    partial = sc_call(packed, plus_req, plus_var, minus_req, minus_var)
    total = jnp.sum(partial)
    denom = jnp.float32(_NUM_REQ * batch)
    return jnp.float32(1.0) - total / denom


# speedup vs baseline: 1.9626x; 1.9626x over previous
import functools

import jax
import jax.numpy as jnp
from jax import lax
from jax.experimental import pallas as pl
from jax.experimental.pallas import tpu as pltpu
from jax.experimental.pallas import tpu_sc as plsc

_NUM_REQ = 512
_LITS = 4
_NC = 2
_NS = 16
_NW = _NC * _NS
_LANES = 16
_CHUNK = 8
_NGROUPS = _NUM_REQ // _LANES


def _body(preds_h, preq_h, pvar_h, mreq_h, mvar_h, out_h,
          preq_v, pvar_v, mreq_v, mvar_v,
          var_v, scale_v, off_v, rows_a, rows_b, sem_a, sem_b, acc_v,
          rows_per_w, num_vars, n_plus, n_minus):
    c = lax.axis_index("c")
    s = lax.axis_index("s")
    wid = s * _NC + c
    base = wid * rows_per_w
    nchunks = rows_per_w // _CHUNK

    if n_plus:
        pltpu.sync_copy(preq_h, preq_v)
        pltpu.sync_copy(pvar_h, pvar_v)
    if n_minus:
        pltpu.sync_copy(mreq_h, mreq_v)
        pltpu.sync_copy(mvar_h, mvar_v)

    iota = lax.iota(jnp.int32, _LANES)
    zero_i = jnp.zeros((_LANES,), jnp.int32)

    def prep(req_v, varr_v, n, is_minus):
        ngrp = (n + _LANES - 1) // _LANES
        nm1 = jnp.full((_LANES,), n - 1, jnp.int32)
        nsplat = jnp.full((_LANES,), n, jnp.int32)
        sc_c = jnp.full((_LANES,), -1.0 if is_minus else 1.0, jnp.float32)
        of_c = jnp.full((_LANES,), 1.0 if is_minus else 0.0, jnp.float32)

        def gbody(g, carry):
            pos = iota + g * _LANES
            cur_i = jnp.minimum(pos, nm1)
            req = plsc.load_gather(req_v, [cur_i])
            var = plsc.load_gather(varr_v, [cur_i])
            k = zero_i
            for t in (1, 2, 3):
                if is_minus:
                    q = pos + t
                    nb_i = jnp.minimum(q, nm1)
                    valid = q < nsplat
                else:
                    q = pos - t
                    nb_i = jnp.maximum(q, zero_i)
                    valid = q >= zero_i
                nb = plsc.load_gather(req_v, [nb_i])
                k = k + jnp.where(valid & (nb == req), 1, 0)
            if is_minus:
                k = 3 - k
            slot = k * _NUM_REQ + req
            mask = pos < nsplat
            plsc.store_scatter(var_v, [slot], var, mask=mask)
            plsc.store_scatter(scale_v, [slot], sc_c, mask=mask)
            plsc.store_scatter(off_v, [slot], of_c, mask=mask)
            return carry

        lax.fori_loop(0, ngrp, gbody, 0)

    if n_plus:
        prep(preq_v, pvar_v, n_plus, False)
    if n_minus:
        prep(mreq_v, mvar_v, n_minus, True)

    def make_g_body(rows_v):
        def g_body(g, accs):
            o = g * _LANES
            idx = [var_v[pl.ds(k * _NUM_REQ + o, _LANES)] for k in range(_LITS)]
            sc = [scale_v[pl.ds(k * _NUM_REQ + o, _LANES)] for k in range(_LITS)]
            of = [off_v[pl.ds(k * _NUM_REQ + o, _LANES)] for k in range(_LITS)]
            accs = list(accs)
            for r in range(_CHUNK):
                rsplat = jnp.full((_LANES,), r, jnp.int32)
                vals = [of[k] + sc[k] * plsc.load_gather(rows_v,
                                                         [rsplat, idx[k]])
                        for k in range(_LITS)]
                m = jnp.maximum(jnp.maximum(vals[0], vals[1]),
                                jnp.maximum(vals[2], vals[3]))
                accs[r % 4] = accs[r % 4] + m
            return tuple(accs)
        return g_body

    def start(ci, dst, sem):
        pltpu.async_copy(preds_h.at[pl.ds(base + ci * _CHUNK, _CHUNK)],
                         dst, sem)

    def wait(dst, sem):
        pltpu.make_async_copy(preds_h.at[pl.ds(0, _CHUNK)], dst, sem).wait()

    last = nchunks - 1
    start(base * 0, rows_a, sem_a)

    def pair_body(p, accs):
        ci1 = 2 * p + 1
        ci2 = jnp.minimum(2 * p + 2, last)
        start(ci1, rows_b, sem_b)
        wait(rows_a, sem_a)
        accs = plsc.parallel_loop(0, _NGROUPS, unroll=2,
                                  carry=tuple(accs))(make_g_body(rows_a))
        start(ci2, rows_a, sem_a)
        wait(rows_b, sem_b)
        accs = plsc.parallel_loop(0, _NGROUPS, unroll=2,
                                  carry=tuple(accs))(make_g_body(rows_b))
        return accs

    z = jnp.zeros((_LANES,), jnp.float32)
    accs = lax.fori_loop(0, nchunks // 2, pair_body, (z, z, z, z))
    wait(rows_a, sem_a)
    acc_v[...] = accs[0] + accs[1] + accs[2] + accs[3]
    pltpu.sync_copy(acc_v, out_h.at[wid])


def kernel(preds, plus_req, plus_var, minus_req, minus_var):
    batch, num_vars = preds.shape
    n_plus = plus_req.shape[0]
    n_minus = minus_req.shape[0]
    tab = _LITS * _NUM_REQ

    rows_per_w = batch // _NW
    mesh = plsc.VectorSubcoreMesh(core_axis_name="c", subcore_axis_name="s")
    sc_call = functools.partial(
        pl.kernel,
        out_type=jax.ShapeDtypeStruct((_NW, _LANES), jnp.float32),
        mesh=mesh,
        compiler_params=pltpu.CompilerParams(needs_layout_passes=False),
        scratch_types=[
            pltpu.VMEM((max(n_plus, 1),), jnp.int32),
            pltpu.VMEM((max(n_plus, 1),), jnp.int32),
            pltpu.VMEM((max(n_minus, 1),), jnp.int32),
            pltpu.VMEM((max(n_minus, 1),), jnp.int32),
            pltpu.VMEM((tab,), jnp.int32),
            pltpu.VMEM((tab,), jnp.float32),
            pltpu.VMEM((tab,), jnp.float32),
            pltpu.VMEM((_CHUNK, num_vars), jnp.float32),
            pltpu.VMEM((_CHUNK, num_vars), jnp.float32),
            pltpu.SemaphoreType.DMA,
            pltpu.SemaphoreType.DMA,
            pltpu.VMEM((_LANES,), jnp.float32),
        ],
    )(functools.partial(_body, rows_per_w=rows_per_w, num_vars=num_vars,
                        n_plus=n_plus, n_minus=n_minus))

    partial = sc_call(preds, plus_req, plus_var, minus_req, minus_var)
    total = jnp.sum(partial)
    denom = jnp.float32(_NUM_REQ * batch)
    return jnp.float32(1.0) - total / denom


# speedup vs baseline: 2.7760x; 1.4144x over previous
import functools

import jax
import jax.numpy as jnp
from jax import lax
from jax.experimental import pallas as pl
from jax.experimental.pallas import tpu as pltpu
from jax.experimental.pallas import tpu_sc as plsc

_NUM_REQ = 512
_LITS = 4
_NC = 2
_NS = 16
_NW = _NC * _NS
_LANES = 16
_CHUNK = 8
_NGROUPS = _NUM_REQ // _LANES


def _body(preds_h, preq_h, pvar_h, mreq_h, mvar_h, out_h,
          preq_v, pvar_v, mreq_v, mvar_v,
          var_v, scale_v, rows_a, rows_b, sem_a, sem_b, acc_v,
          rows_per_w, num_vars, n_plus, n_minus):
    c = lax.axis_index("c")
    s = lax.axis_index("s")
    wid = s * _NC + c
    base = wid * rows_per_w
    nchunks = rows_per_w // _CHUNK

    if n_plus:
        pltpu.sync_copy(preq_h, preq_v)
        pltpu.sync_copy(pvar_h, pvar_v)
    if n_minus:
        pltpu.sync_copy(mreq_h, mreq_v)
        pltpu.sync_copy(mvar_h, mvar_v)

    iota = lax.iota(jnp.int32, _LANES)
    zero_i = jnp.zeros((_LANES,), jnp.int32)

    def prep(req_v, varr_v, n, is_minus):
        ngrp = (n + _LANES - 1) // _LANES
        nm1 = jnp.full((_LANES,), n - 1, jnp.int32)
        nsplat = jnp.full((_LANES,), n, jnp.int32)
        sc_c = jnp.full((_LANES,), -1.0 if is_minus else 1.0, jnp.float32)

        def gbody(g, carry):
            pos = iota + g * _LANES
            cur_i = jnp.minimum(pos, nm1)
            req = plsc.load_gather(req_v, [cur_i])
            var = plsc.load_gather(varr_v, [cur_i])
            k = zero_i
            for t in (1, 2, 3):
                if is_minus:
                    q = pos + t
                    nb_i = jnp.minimum(q, nm1)
                    valid = q < nsplat
                else:
                    q = pos - t
                    nb_i = jnp.maximum(q, zero_i)
                    valid = q >= zero_i
                nb = plsc.load_gather(req_v, [nb_i])
                k = k + jnp.where(valid & (nb == req), 1, 0)
            if is_minus:
                k = 3 - k
            slot = k * _NUM_REQ + req
            mask = pos < nsplat
            plsc.store_scatter(var_v, [slot], var, mask=mask)
            plsc.store_scatter(scale_v, [slot], sc_c, mask=mask)
            return carry

        lax.fori_loop(0, ngrp, gbody, 0)

    if n_plus:
        prep(preq_v, pvar_v, n_plus, False)
    if n_minus:
        prep(mreq_v, mvar_v, n_minus, True)

    def make_g_body(rows_v):
        half = jnp.full((_LANES,), 0.5, jnp.float32)

        def g_body(g, accs):
            o = g * _LANES
            idx = [var_v[pl.ds(k * _NUM_REQ + o, _LANES)] for k in range(_LITS)]
            sc = [scale_v[pl.ds(k * _NUM_REQ + o, _LANES)] for k in range(_LITS)]
            accs = list(accs)
            for r in range(_CHUNK):
                rsplat = jnp.full((_LANES,), r, jnp.int32)
                vals = [sc[k] * (plsc.load_gather(rows_v, [rsplat, idx[k]])
                                 - half)
                        for k in range(_LITS)]
                m = jnp.maximum(jnp.maximum(vals[0], vals[1]),
                                jnp.maximum(vals[2], vals[3]))
                accs[r % 4] = accs[r % 4] + m
            return tuple(accs)
        return g_body

    def start(ci, dst, sem):
        pltpu.async_copy(preds_h.at[pl.ds(base + ci * _CHUNK, _CHUNK)],
                         dst, sem)

    def wait(dst, sem):
        pltpu.make_async_copy(preds_h.at[pl.ds(0, _CHUNK)], dst, sem).wait()

    last = nchunks - 1
    start(base * 0, rows_a, sem_a)

    def pair_body(p, accs):
        ci1 = 2 * p + 1
        ci2 = jnp.minimum(2 * p + 2, last)
        start(ci1, rows_b, sem_b)
        wait(rows_a, sem_a)
        accs = lax.fori_loop(0, _NGROUPS, make_g_body(rows_a), accs)
        start(ci2, rows_a, sem_a)
        wait(rows_b, sem_b)
        accs = lax.fori_loop(0, _NGROUPS, make_g_body(rows_b), accs)
        return accs

    z = jnp.zeros((_LANES,), jnp.float32)
    accs = lax.fori_loop(0, nchunks // 2, pair_body, (z, z, z, z))
    wait(rows_a, sem_a)
    acc_v[...] = accs[0] + accs[1] + accs[2] + accs[3]
    pltpu.sync_copy(acc_v, out_h.at[wid])


def kernel(preds, plus_req, plus_var, minus_req, minus_var):
    batch, num_vars = preds.shape
    n_plus = plus_req.shape[0]
    n_minus = minus_req.shape[0]
    tab = _LITS * _NUM_REQ

    rows_per_w = batch // _NW
    mesh = plsc.VectorSubcoreMesh(core_axis_name="c", subcore_axis_name="s")
    sc_call = functools.partial(
        pl.kernel,
        out_type=jax.ShapeDtypeStruct((_NW, _LANES), jnp.float32),
        mesh=mesh,
        compiler_params=pltpu.CompilerParams(needs_layout_passes=False),
        scratch_types=[
            pltpu.VMEM((max(n_plus, 1),), jnp.int32),
            pltpu.VMEM((max(n_plus, 1),), jnp.int32),
            pltpu.VMEM((max(n_minus, 1),), jnp.int32),
            pltpu.VMEM((max(n_minus, 1),), jnp.int32),
            pltpu.VMEM((tab,), jnp.int32),
            pltpu.VMEM((tab,), jnp.float32),
            pltpu.VMEM((_CHUNK, num_vars), jnp.float32),
            pltpu.VMEM((_CHUNK, num_vars), jnp.float32),
            pltpu.SemaphoreType.DMA,
            pltpu.SemaphoreType.DMA,
            pltpu.VMEM((_LANES,), jnp.float32),
        ],
    )(functools.partial(_body, rows_per_w=rows_per_w, num_vars=num_vars,
                        n_plus=n_plus, n_minus=n_minus))

    partial = sc_call(preds, plus_req, plus_var, minus_req, minus_var)
    total = jnp.sum(partial)
    denom = jnp.float32(_NUM_REQ * batch)
    return jnp.float32(0.5) - total / denom


# speedup vs baseline: 2.8098x; 1.0122x over previous
import functools

import jax
import jax.numpy as jnp
from jax import lax
from jax.experimental import pallas as pl
from jax.experimental.pallas import tpu as pltpu
from jax.experimental.pallas import tpu_sc as plsc

_NUM_REQ = 512
_LITS = 4
_NC = 2
_NS = 16
_NW = _NC * _NS
_LANES = 16
_CHUNK = 8
_NGROUPS = _NUM_REQ // _LANES


def _body(preds_h, preq_h, pvar_h, mreq_h, mvar_h, out_h,
          preq_v, pvar_v, mreq_v, mvar_v,
          var_v, scale_v, rows_a, rows_b, sem_a, sem_b, sem_t, acc_v,
          rows_per_w, num_vars, n_plus, n_minus):
    c = lax.axis_index("c")
    s = lax.axis_index("s")
    wid = s * _NC + c
    base = wid * rows_per_w
    nchunks = rows_per_w // _CHUNK

    def start(ci, dst, sem):
        pltpu.async_copy(preds_h.at[pl.ds(base + ci * _CHUNK, _CHUNK)],
                         dst, sem)

    def wait(dst, sem):
        pltpu.make_async_copy(preds_h.at[pl.ds(0, _CHUNK)], dst, sem).wait()

    start(0, rows_a, sem_a)
    start(1, rows_b, sem_b)
    lists = []
    if n_plus:
        lists += [(preq_h, preq_v), (pvar_h, pvar_v)]
    if n_minus:
        lists += [(mreq_h, mreq_v), (mvar_h, mvar_v)]
    for src, dst in lists:
        pltpu.async_copy(src, dst, sem_t)
    for src, dst in lists:
        pltpu.make_async_copy(src, dst, sem_t).wait()

    iota = lax.iota(jnp.int32, _LANES)
    zero_i = jnp.zeros((_LANES,), jnp.int32)

    def prep(req_v, varr_v, n, is_minus):
        ngrp = (n + _LANES - 1) // _LANES
        nm1 = jnp.full((_LANES,), n - 1, jnp.int32)
        nsplat = jnp.full((_LANES,), n, jnp.int32)
        sc_c = jnp.full((_LANES,), -1.0 if is_minus else 1.0, jnp.float32)

        def gbody(g, carry):
            pos = iota + g * _LANES
            cur_i = jnp.minimum(pos, nm1)
            req = plsc.load_gather(req_v, [cur_i])
            var = plsc.load_gather(varr_v, [cur_i])
            k = zero_i
            for t in (1, 2, 3):
                if is_minus:
                    q = pos + t
                    nb_i = jnp.minimum(q, nm1)
                    valid = q < nsplat
                else:
                    q = pos - t
                    nb_i = jnp.maximum(q, zero_i)
                    valid = q >= zero_i
                nb = plsc.load_gather(req_v, [nb_i])
                k = k + jnp.where(valid & (nb == req), 1, 0)
            if is_minus:
                k = 3 - k
            slot = k * _NUM_REQ + req
            mask = pos < nsplat
            plsc.store_scatter(var_v, [slot], var, mask=mask)
            plsc.store_scatter(scale_v, [slot], sc_c, mask=mask)
            return carry

        lax.fori_loop(0, ngrp, gbody, 0, unroll=2)

    if n_plus:
        prep(preq_v, pvar_v, n_plus, False)
    if n_minus:
        prep(mreq_v, mvar_v, n_minus, True)

    def make_g_body(rows_v):
        half = jnp.full((_LANES,), 0.5, jnp.float32)

        def g_body(g, accs):
            o = g * _LANES
            idx = [var_v[pl.ds(k * _NUM_REQ + o, _LANES)] for k in range(_LITS)]
            sc = [scale_v[pl.ds(k * _NUM_REQ + o, _LANES)] for k in range(_LITS)]
            accs = list(accs)
            for r in range(_CHUNK):
                rsplat = jnp.full((_LANES,), r, jnp.int32)
                vals = [sc[k] * (plsc.load_gather(rows_v, [rsplat, idx[k]])
                                 - half)
                        for k in range(_LITS)]
                m = jnp.maximum(jnp.maximum(vals[0], vals[1]),
                                jnp.maximum(vals[2], vals[3]))
                accs[r % 4] = accs[r % 4] + m
            return tuple(accs)
        return g_body

    last = nchunks - 1

    def pair_body(p, accs):
        ci2 = jnp.minimum(2 * p + 2, last)
        ci3 = jnp.minimum(2 * p + 3, last)
        wait(rows_a, sem_a)
        accs = lax.fori_loop(0, _NGROUPS, make_g_body(rows_a), accs)
        start(ci2, rows_a, sem_a)
        wait(rows_b, sem_b)
        accs = lax.fori_loop(0, _NGROUPS, make_g_body(rows_b), accs)
        start(ci3, rows_b, sem_b)
        return accs

    z = jnp.zeros((_LANES,), jnp.float32)
    accs = lax.fori_loop(0, nchunks // 2, pair_body, (z, z, z, z))
    wait(rows_a, sem_a)
    wait(rows_b, sem_b)
    acc_v[...] = accs[0] + accs[1] + accs[2] + accs[3]
    pltpu.sync_copy(acc_v, out_h.at[wid])


def kernel(preds, plus_req, plus_var, minus_req, minus_var):
    batch, num_vars = preds.shape
    n_plus = plus_req.shape[0]
    n_minus = minus_req.shape[0]
    tab = _LITS * _NUM_REQ

    rows_per_w = batch // _NW
    mesh = plsc.VectorSubcoreMesh(core_axis_name="c", subcore_axis_name="s")
    sc_call = functools.partial(
        pl.kernel,
        out_type=jax.ShapeDtypeStruct((_NW, _LANES), jnp.float32),
        mesh=mesh,
        compiler_params=pltpu.CompilerParams(needs_layout_passes=False),
        scratch_types=[
            pltpu.VMEM((max(n_plus, 1),), jnp.int32),
            pltpu.VMEM((max(n_plus, 1),), jnp.int32),
            pltpu.VMEM((max(n_minus, 1),), jnp.int32),
            pltpu.VMEM((max(n_minus, 1),), jnp.int32),
            pltpu.VMEM((tab,), jnp.int32),
            pltpu.VMEM((tab,), jnp.float32),
            pltpu.VMEM((_CHUNK, num_vars), jnp.float32),
            pltpu.VMEM((_CHUNK, num_vars), jnp.float32),
            pltpu.SemaphoreType.DMA,
            pltpu.SemaphoreType.DMA,
            pltpu.SemaphoreType.DMA,
            pltpu.VMEM((_LANES,), jnp.float32),
        ],
    )(functools.partial(_body, rows_per_w=rows_per_w, num_vars=num_vars,
                        n_plus=n_plus, n_minus=n_minus))

    partial = sc_call(preds, plus_req, plus_var, minus_req, minus_var)
    total = jnp.sum(partial)
    denom = jnp.float32(_NUM_REQ * batch)
    return jnp.float32(0.5) - total / denom
